# baseline (speedup 1.0000x reference)
import functools

import jax
import jax.numpy as jnp
from jax import lax
from jax.experimental import pallas as pl
from jax.experimental.pallas import tpu as pltpu
from jax.experimental.pallas import tpu_sc as plsc

B = 16384
D_EMB = 32
_N_SCHOOL = 102
_N_GOAL = 22
_N_METHOD = 12

_NC = 2
_NS = 16
_NW = _NC * _NS
_BPW = B // _NW
_L = 16

_TC_BLK = 8192


def _sc_gather_body(sidx, gidx, midx, tab, out,
                    tab_v, idx_v, rows_v, tsem, isem, wsem):
    wid = lax.axis_index("s") * _NC + lax.axis_index("c")
    base = wid * _BPW
    idxs = (sidx, gidx, midx)
    copies = [pltpu.async_copy(tab, tab_v, tsem)]
    copies += [pltpu.async_copy(idxs[t].at[pl.ds(base, _BPW)],
                                idx_v.at[t], isem) for t in range(3)]
    for c in copies:
        c.wait()
    iota16 = lax.iota(jnp.int32, _L)
    colvs = [jnp.bitwise_and(c + iota16, D_EMB - 1) for c in range(D_EMB)]

    tab_base = (0, _N_SCHOOL * D_EMB, (_N_SCHOOL + _N_GOAL) * D_EMB)

    @plsc.parallel_loop(0, _BPW // _L, unroll=2)
    def _lookup(g):
        rowv = g * _L + iota16
        for t in range(3):
            iv = idx_v[t, pl.ds(g * _L, _L)]
            off = iv * D_EMB + tab_base[t]
            for c in range(D_EMB):
                vals = plsc.load_gather(tab_v, [off + colvs[c]])
                plsc.store_scatter(rows_v, [rowv, colvs[c] + t * D_EMB],
                                   vals)
    pltpu.async_copy(rows_v, out.at[pl.ds(base, _BPW)], wsem).wait()


@jax.jit
def _sc_gather(school_idx, goal_idx, method_idx, tab_flat):
    mesh = plsc.VectorSubcoreMesh(core_axis_name="c", subcore_axis_name="s")
    n_tab = (_N_SCHOOL + _N_GOAL + _N_METHOD) * D_EMB
    return pl.kernel(
        _sc_gather_body,
        out_type=jax.ShapeDtypeStruct((B, 4 * D_EMB), jnp.float32),
        mesh=mesh,
        scratch_types=[
            pltpu.VMEM((n_tab,), jnp.float32),
            pltpu.VMEM((3, _BPW), jnp.int32),
            pltpu.VMEM((_BPW, 4 * D_EMB), jnp.float32),
            pltpu.SemaphoreType.DMA,
            pltpu.SemaphoreType.DMA,
            pltpu.SemaphoreType.DMA,
        ],
        compiler_params=pltpu.CompilerParams(use_tc_tiling_on_sc=False,
                                             needs_layout_passes=False),
    )(school_idx, goal_idx, method_idx, tab_flat)


def _tc_mlp_body(e128, sf, gf, wsub, bsub, wgrd, bgrd,
                 w1, b1, w2, b2, w3, b3, out):
    f32 = jnp.float32
    dot = functools.partial(jnp.dot, preferred_element_type=f32)
    w1_all = w1[...]
    e = e128[...]
    col = lax.broadcasted_iota(jnp.int32, e.shape, 1)
    e = jnp.where(col < 3 * D_EMB, e, 0.0)
    subj = dot(sf[...], wsub[...]) + bsub[...]
    grd = dot(gf[...], wgrd[...]) + bgrd[...]
    x = (dot(e, w1_all[0:128])
         + dot(subj, w1_all[96:128])
         + dot(grd, w1_all[128:160])
         + b1[...])
    h = jnp.maximum(x, 0.0)
    h = jnp.maximum(dot(h, w2[...]) + b2[...], 0.0)
    out[...] = dot(h, w3[...]) + b3[...]


@jax.jit
def _tc_mlp(e128, subject_feats, grade_feats,
            W_subj, b_subj, W_grade, b_grade, W1, b1, W2, b2, W3, b3):
    nblk = B // _TC_BLK
    row = lambda i: (i, 0)
    rep = lambda i: (0, 0)

    def spec(shape, index_map):
        return pl.BlockSpec(shape, index_map)

    return pl.pallas_call(
        _tc_mlp_body,
        grid=(nblk,),
        in_specs=[
            spec((_TC_BLK, 128), row),
            spec((_TC_BLK, 10), row),
            spec((_TC_BLK, 12), row),
            spec((10, 32), rep),
            spec((1, 32), rep),
            spec((12, 32), rep),
            spec((1, 32), rep),
            spec((160, 128), rep),
            spec((1, 128), rep),
            spec((128, 64), rep),
            spec((1, 64), rep),
            spec((64, 32), rep),
            spec((1, 32), rep),
        ],
        out_specs=spec((_TC_BLK, 32), row),
        out_shape=jax.ShapeDtypeStruct((B, 32), jnp.float32),
    )(e128, subject_feats, grade_feats,
      W_subj, b_subj, W_grade, b_grade, W1, b1, W2, b2, W3, b3)


def kernel(school_idx, goal_idx, method_idx, subject_feats, grade_feats,
           school_table, goal_table, method_table,
           W_subj, b_subj, W_grade, b_grade, W1, b1, W2, b2, W3, b3):
    tab_flat = jnp.concatenate([school_table.reshape(-1),
                                goal_table.reshape(-1),
                                method_table.reshape(-1)])
    e128 = _sc_gather(school_idx, goal_idx, method_idx, tab_flat)
    return _tc_mlp(e128, subject_feats, grade_feats,
                   W_subj, b_subj.reshape(1, -1), W_grade,
                   b_grade.reshape(1, -1), W1, b1.reshape(1, -1),
                   W2, b2.reshape(1, -1), W3, b3.reshape(1, -1))


# speedup vs baseline: 1.0249x; 1.0249x over previous
import functools

import jax
import jax.numpy as jnp
from jax import lax
from jax.experimental import pallas as pl
from jax.experimental.pallas import tpu as pltpu
from jax.experimental.pallas import tpu_sc as plsc

B = 16384
D_EMB = 32
_N_SCHOOL = 102
_N_GOAL = 22
_N_METHOD = 12

_NC = 2
_NS = 16
_NW = _NC * _NS
_BPW = B // _NW
_L = 16

_TC_BLK = 4096


def _sc_gather_body(sidx, gidx, midx, tab, out,
                    tab_v, idx_v, rows_v, tsem, isem, wsem):
    wid = lax.axis_index("s") * _NC + lax.axis_index("c")
    base = wid * _BPW
    idxs = (sidx, gidx, midx)
    copies = [pltpu.async_copy(tab, tab_v, tsem)]
    copies += [pltpu.async_copy(idxs[t].at[pl.ds(base, _BPW)],
                                idx_v.at[t], isem) for t in range(3)]
    for c in copies:
        c.wait()
    iota16 = lax.iota(jnp.int32, _L)
    colvs = [jnp.bitwise_and(c + iota16, D_EMB - 1) for c in range(D_EMB)]

    tab_base = (0, _N_SCHOOL * D_EMB, (_N_SCHOOL + _N_GOAL) * D_EMB)

    @plsc.parallel_loop(0, _BPW // _L, unroll=2)
    def _lookup(g):
        rowv = g * _L + iota16
        for t in range(3):
            iv = idx_v[t, pl.ds(g * _L, _L)]
            off = iv * D_EMB + tab_base[t]
            for c in range(D_EMB):
                vals = plsc.load_gather(tab_v, [off + colvs[c]])
                plsc.store_scatter(rows_v, [rowv, colvs[c] + t * D_EMB],
                                   vals)
    pltpu.async_copy(rows_v, out.at[pl.ds(base, _BPW)], wsem).wait()


@jax.jit
def _sc_gather(school_idx, goal_idx, method_idx, tab_flat):
    mesh = plsc.VectorSubcoreMesh(core_axis_name="c", subcore_axis_name="s")
    n_tab = (_N_SCHOOL + _N_GOAL + _N_METHOD) * D_EMB
    return pl.kernel(
        _sc_gather_body,
        out_type=jax.ShapeDtypeStruct((B, 4 * D_EMB), jnp.float32),
        mesh=mesh,
        scratch_types=[
            pltpu.VMEM((n_tab,), jnp.float32),
            pltpu.VMEM((3, _BPW), jnp.int32),
            pltpu.VMEM((_BPW, 4 * D_EMB), jnp.float32),
            pltpu.SemaphoreType.DMA,
            pltpu.SemaphoreType.DMA,
            pltpu.SemaphoreType.DMA,
        ],
        compiler_params=pltpu.CompilerParams(use_tc_tiling_on_sc=False,
                                             needs_layout_passes=False),
    )(school_idx, goal_idx, method_idx, tab_flat)


def _tc_mlp_body(e128, sf, gf, wsub, bsub, wgrd, bgrd,
                 w1, b1, w2, b2, w3, b3, out):
    f32 = jnp.float32
    dot = functools.partial(jnp.dot, preferred_element_type=f32)
    w1_all = w1[...]
    e = e128[...]
    col = lax.broadcasted_iota(jnp.int32, e.shape, 1)
    e = jnp.where(col < 3 * D_EMB, e, 0.0)
    subj = dot(sf[...], wsub[...]) + bsub[...]
    grd = dot(gf[...], wgrd[...]) + bgrd[...]
    x = (dot(e, w1_all[0:128])
         + dot(subj, w1_all[96:128])
         + dot(grd, w1_all[128:160])
         + b1[...])
    h = jnp.maximum(x, 0.0)
    h = jnp.maximum(dot(h, w2[...]) + b2[...], 0.0)
    out[...] = dot(h, w3[...]) + b3[...]


@jax.jit
def _tc_mlp(e128, subject_feats, grade_feats,
            W_subj, b_subj, W_grade, b_grade, W1, b1, W2, b2, W3, b3):
    nblk = B // _TC_BLK
    row = lambda i: (i, 0)
    rep = lambda i: (0, 0)

    def spec(shape, index_map):
        return pl.BlockSpec(shape, index_map)

    return pl.pallas_call(
        _tc_mlp_body,
        grid=(nblk,),
        in_specs=[
            spec((_TC_BLK, 128), row),
            spec((_TC_BLK, 10), row),
            spec((_TC_BLK, 12), row),
            spec((10, 32), rep),
            spec((1, 32), rep),
            spec((12, 32), rep),
            spec((1, 32), rep),
            spec((160, 128), rep),
            spec((1, 128), rep),
            spec((128, 64), rep),
            spec((1, 64), rep),
            spec((64, 32), rep),
            spec((1, 32), rep),
        ],
        out_specs=spec((_TC_BLK, 32), row),
        out_shape=jax.ShapeDtypeStruct((B, 32), jnp.float32),
        compiler_params=pltpu.CompilerParams(
            dimension_semantics=("arbitrary",)),
    )(e128, subject_feats, grade_feats,
      W_subj, b_subj, W_grade, b_grade, W1, b1, W2, b2, W3, b3)


def kernel(school_idx, goal_idx, method_idx, subject_feats, grade_feats,
           school_table, goal_table, method_table,
           W_subj, b_subj, W_grade, b_grade, W1, b1, W2, b2, W3, b3):
    tab_flat = jnp.concatenate([school_table.reshape(-1),
                                goal_table.reshape(-1),
                                method_table.reshape(-1)])
    e128 = _sc_gather(school_idx, goal_idx, method_idx, tab_flat)
    return _tc_mlp(e128, subject_feats, grade_feats,
                   W_subj, b_subj.reshape(1, -1), W_grade,
                   b_grade.reshape(1, -1), W1, b1.reshape(1, -1),
                   W2, b2.reshape(1, -1), W3, b3.reshape(1, -1))


# speedup vs baseline: 1.0262x; 1.0013x over previous
import functools

import jax
import jax.numpy as jnp
from jax import lax
from jax.experimental import pallas as pl
from jax.experimental.pallas import tpu as pltpu
from jax.experimental.pallas import tpu_sc as plsc

B = 16384
D_EMB = 32
_N_SCHOOL = 102
_N_GOAL = 22
_N_METHOD = 12

_NC = 2
_NS = 16
_NW = _NC * _NS
_BPW = B // _NW
_L = 16

_TC_BLK = 4096


def _sc_gather_body(sidx, gidx, midx, tab, out,
                    tab_v, idx_v, rows_v, tsem, isem, wsem):
    wid = lax.axis_index("s") * _NC + lax.axis_index("c")
    base = wid * _BPW
    idxs = (sidx, gidx, midx)
    copies = [pltpu.async_copy(tab, tab_v, tsem)]
    copies += [pltpu.async_copy(idxs[t].at[pl.ds(base, _BPW)],
                                idx_v.at[t], isem) for t in range(3)]
    for c in copies:
        c.wait()
    iota16 = lax.iota(jnp.int32, _L)
    colvs = [jnp.bitwise_and(c + iota16, D_EMB - 1) for c in range(D_EMB)]

    tab_base = (0, _N_SCHOOL * D_EMB, (_N_SCHOOL + _N_GOAL) * D_EMB)

    @plsc.parallel_loop(0, _BPW // _L, unroll=2)
    def _lookup(g):
        rowv = g * _L + iota16
        for t in range(3):
            iv = idx_v[t, pl.ds(g * _L, _L)]
            off = iv * D_EMB + tab_base[t]
            for c in range(D_EMB):
                vals = plsc.load_gather(tab_v, [off + colvs[c]])
                plsc.store_scatter(rows_v, [rowv, colvs[c] + t * D_EMB],
                                   vals)
    pltpu.async_copy(rows_v, out.at[pl.ds(base, _BPW)], wsem).wait()


@jax.jit
def _sc_gather(school_idx, goal_idx, method_idx, tab_flat):
    mesh = plsc.VectorSubcoreMesh(core_axis_name="c", subcore_axis_name="s")
    n_tab = (_N_SCHOOL + _N_GOAL + _N_METHOD) * D_EMB
    return pl.kernel(
        _sc_gather_body,
        out_type=jax.ShapeDtypeStruct((B, 4 * D_EMB), jnp.float32),
        mesh=mesh,
        scratch_types=[
            pltpu.VMEM((n_tab,), jnp.float32),
            pltpu.VMEM((3, _BPW), jnp.int32),
            pltpu.VMEM((_BPW, 4 * D_EMB), jnp.float32),
            pltpu.SemaphoreType.DMA,
            pltpu.SemaphoreType.DMA,
            pltpu.SemaphoreType.DMA,
        ],
        compiler_params=pltpu.CompilerParams(use_tc_tiling_on_sc=False,
                                             needs_layout_passes=False),
    )(school_idx, goal_idx, method_idx, tab_flat)


def _tc_mlp_body(e128, sf, gf, wsub, bsub, wgrd, bgrd,
                 w1, b1, w2, b2, w3, b3, out):
    f32 = jnp.float32
    dot = functools.partial(jnp.dot, preferred_element_type=f32)
    w1_all = w1[...]
    e = e128[...]
    wrow = lax.broadcasted_iota(jnp.int32, (128, 128), 0)
    w1e = jnp.where(wrow < 3 * D_EMB, w1_all[0:128], 0.0)
    subj = dot(sf[...], wsub[...]) + bsub[...]
    grd = dot(gf[...], wgrd[...]) + bgrd[...]
    x = (dot(e, w1e)
         + dot(subj, w1_all[96:128])
         + dot(grd, w1_all[128:160])
         + b1[...])
    h = jnp.maximum(x, 0.0)
    h = jnp.maximum(dot(h, w2[...]) + b2[...], 0.0)
    out[...] = dot(h, w3[...]) + b3[...]


@jax.jit
def _tc_mlp(e128, subject_feats, grade_feats,
            W_subj, b_subj, W_grade, b_grade, W1, b1, W2, b2, W3, b3):
    nblk = B // _TC_BLK
    row = lambda i: (i, 0)
    rep = lambda i: (0, 0)

    def spec(shape, index_map):
        return pl.BlockSpec(shape, index_map)

    return pl.pallas_call(
        _tc_mlp_body,
        grid=(nblk,),
        in_specs=[
            spec((_TC_BLK, 128), row),
            spec((_TC_BLK, 10), row),
            spec((_TC_BLK, 12), row),
            spec((10, 32), rep),
            spec((1, 32), rep),
            spec((12, 32), rep),
            spec((1, 32), rep),
            spec((160, 128), rep),
            spec((1, 128), rep),
            spec((128, 64), rep),
            spec((1, 64), rep),
            spec((64, 32), rep),
            spec((1, 32), rep),
        ],
        out_specs=spec((_TC_BLK, 32), row),
        out_shape=jax.ShapeDtypeStruct((B, 32), jnp.float32),
        compiler_params=pltpu.CompilerParams(
            dimension_semantics=("arbitrary",)),
    )(e128, subject_feats, grade_feats,
      W_subj, b_subj, W_grade, b_grade, W1, b1, W2, b2, W3, b3)


def kernel(school_idx, goal_idx, method_idx, subject_feats, grade_feats,
           school_table, goal_table, method_table,
           W_subj, b_subj, W_grade, b_grade, W1, b1, W2, b2, W3, b3):
    tab_flat = jnp.concatenate([school_table.reshape(-1),
                                goal_table.reshape(-1),
                                method_table.reshape(-1)])
    e128 = _sc_gather(school_idx, goal_idx, method_idx, tab_flat)
    return _tc_mlp(e128, subject_feats, grade_feats,
                   W_subj, b_subj.reshape(1, -1), W_grade,
                   b_grade.reshape(1, -1), W1, b1.reshape(1, -1),
                   W2, b2.reshape(1, -1), W3, b3.reshape(1, -1))
